# Initial kernel scaffold; baseline (speedup 1.0000x reference)
import jax
import jax.numpy as jnp
from jax.experimental import pallas as pl


def kernel(data, true_branch_nodes, false_branch_nodes, start_index, exit_index, steps, embed, Wi, Wh, b, W_bd, b_bd, W_out, b_out):
    raise NotImplementedError("write your pallas kernel here")


# speedup vs baseline: 40.4617x; 40.4617x over previous
import jax
import jax.numpy as jnp
from jax.experimental import pallas as pl
from jax.experimental.pallas import tpu as pltpu

_B = 8
_N = 512
_L = 4
_H = 64
_VOCAB = 1024
_OUT = 1000
_LAYERS = 2
_STEPS = 4

_F32 = jnp.float32


def _mm(a, b):
    return jax.lax.dot_general(a, b, (((1,), (0,)), ((), ())),
                               preferred_element_type=_F32)


def _mm_t(a, b):
    return jax.lax.dot_general(a, b, (((0,), (0,)), ((), ())),
                               preferred_element_type=_F32)


def _fwd_kernel(si_ref, ei_ref, bd_b_ref,
                data_ref, ti_ref, fi_ref,
                embed_ref, wi_ref, wh_ref, b_ref, wd_ref, wout_ref, bout_ref,
                out_ref):
    pid = pl.program_id(0)
    si = si_ref[pid]
    ei = ei_ref[pid]

    iota_col = jax.lax.broadcasted_iota(jnp.int32, (_N, 1), 0)
    ip = (iota_col == si).astype(_F32)
    exit_mask = iota_col == ei

    iota_vocab = jax.lax.broadcasted_iota(jnp.int32, (_N, _VOCAB), 1)
    xs = []
    for tok in range(_L):
        dcol = data_ref[0, :, tok:tok + 1]
        oh = (dcol == iota_vocab).astype(_F32)
        xs.append(_mm(oh, embed_ref[...]))

    ti_col = ti_ref[0]
    fi_col = fi_ref[0]
    iota_row = jax.lax.broadcasted_iota(jnp.int32, (_N, _N), 1)
    t_oh = (ti_col == iota_row).astype(_F32)
    f_oh = (fi_col == iota_row).astype(_F32)

    ones_col = jnp.ones((_N, 1), _F32)

    c = [jnp.zeros((_N, _H), _F32) for _ in range(_LAYERS)]
    h = [jnp.zeros((_N, _H), _F32) for _ in range(_LAYERS)]

    for _step in range(_STEPS):
        cc = list(c)
        hh = list(h)
        for tok in range(_L):
            inp = xs[tok]
            for lay in range(_LAYERS):
                gates = (_mm(inp, wi_ref[lay]) + _mm(hh[lay], wh_ref[lay])
                         + b_ref[lay])
                ig = jax.nn.sigmoid(gates[:, 0 * _H:1 * _H])
                fg = jax.nn.sigmoid(gates[:, 1 * _H:2 * _H])
                gg = jnp.tanh(gates[:, 2 * _H:3 * _H])
                og = jax.nn.sigmoid(gates[:, 3 * _H:4 * _H])
                cc[lay] = fg * cc[lay] + ig * gg
                hh[lay] = og * jnp.tanh(cc[lay])
                inp = hh[lay]
        cm = [jnp.where(exit_mask, c[lay], cc[lay]) for lay in range(_LAYERS)]
        hm = [jnp.where(exit_mask, h[lay], hh[lay]) for lay in range(_LAYERS)]

        d = (_mm(cm[0], wd_ref[0 * _H:1 * _H]) + _mm(cm[1], wd_ref[1 * _H:2 * _H])
             + _mm(hm[0], wd_ref[2 * _H:3 * _H]) + _mm(hm[1], wd_ref[3 * _H:4 * _H])
             + bd_b_ref[0])
        pt = jax.nn.sigmoid(d) * ip
        pf = ip - pt

        bm = t_oh * pt + f_oh * pf
        denom = _mm_t(bm, ones_col)
        inv = 1.0 / (denom + 1e-7)
        c = [_mm_t(bm, cm[lay]) * inv for lay in range(_LAYERS)]
        h = [_mm_t(bm, hm[lay]) * inv for lay in range(_LAYERS)]
        ip = denom

    mf = exit_mask.astype(_F32)
    ec = [_mm_t(mf, c[lay]) for lay in range(_LAYERS)]
    eh = [_mm_t(mf, h[lay]) for lay in range(_LAYERS)]
    out = (_mm(ec[0], wout_ref[0 * _H:1 * _H]) + _mm(ec[1], wout_ref[1 * _H:2 * _H])
           + _mm(eh[0], wout_ref[2 * _H:3 * _H]) + _mm(eh[1], wout_ref[3 * _H:4 * _H])
           + bout_ref[...])
    out_ref[0] = out


def kernel(data, true_branch_nodes, false_branch_nodes, start_index, exit_index,
           steps, embed, Wi, Wh, b, W_bd, b_bd, W_out, b_out):
    del steps
    ti3 = true_branch_nodes.astype(jnp.int32).reshape(_B, _N, 1)
    fi3 = false_branch_nodes.astype(jnp.int32).reshape(_B, _N, 1)
    wd = (W_bd[:, 0] - W_bd[:, 1]).reshape(4 * _H, 1)
    bd_b = (b_bd[0] - b_bd[1]).reshape(1)
    b2 = b.reshape(_LAYERS, 1, 4 * _H)
    bout2 = b_out.reshape(1, _OUT)
    si = start_index.astype(jnp.int32)
    ei = exit_index.astype(jnp.int32)

    out = pl.pallas_call(
        _fwd_kernel,
        grid=(_B,),
        in_specs=[
            pl.BlockSpec(memory_space=pltpu.SMEM),
            pl.BlockSpec(memory_space=pltpu.SMEM),
            pl.BlockSpec(memory_space=pltpu.SMEM),
            pl.BlockSpec((1, _N, _L), lambda bb: (bb, 0, 0)),
            pl.BlockSpec((1, _N, 1), lambda bb: (bb, 0, 0)),
            pl.BlockSpec((1, _N, 1), lambda bb: (bb, 0, 0)),
            pl.BlockSpec((_VOCAB, _H), lambda bb: (0, 0)),
            pl.BlockSpec((_LAYERS, _H, 4 * _H), lambda bb: (0, 0, 0)),
            pl.BlockSpec((_LAYERS, _H, 4 * _H), lambda bb: (0, 0, 0)),
            pl.BlockSpec((_LAYERS, 1, 4 * _H), lambda bb: (0, 0, 0)),
            pl.BlockSpec((4 * _H, 1), lambda bb: (0, 0)),
            pl.BlockSpec((4 * _H, _OUT), lambda bb: (0, 0)),
            pl.BlockSpec((1, _OUT), lambda bb: (0, 0)),
        ],
        out_specs=pl.BlockSpec((1, 1, _OUT), lambda bb: (bb, 0, 0)),
        out_shape=jax.ShapeDtypeStruct((_B, 1, _OUT), _F32),
        compiler_params=pltpu.CompilerParams(
            dimension_semantics=("arbitrary",)),
    )(si, ei, bd_b, data, ti3, fi3, embed, Wi, Wh, b2, wd, W_out, bout2)
    return out.reshape(_B, _OUT)


# speedup vs baseline: 51.7784x; 1.2797x over previous
import jax
import jax.numpy as jnp
from jax.experimental import pallas as pl
from jax.experimental.pallas import tpu as pltpu

_B = 8
_N = 512
_L = 4
_H = 64
_VOCAB = 1024
_OUT = 1000
_LAYERS = 2
_STEPS = 4

_F32 = jnp.float32


def _mm(a, b):
    return jax.lax.dot_general(a, b, (((1,), (0,)), ((), ())),
                               preferred_element_type=_F32)


def _mm_t(a, b):
    return jax.lax.dot_general(a, b, (((0,), (0,)), ((), ())),
                               preferred_element_type=_F32)


def _sigmoid(x):
    return 0.5 * jnp.tanh(0.5 * x) + 0.5


def _fwd_kernel(si_ref, ei_ref, bd_b_ref,
                data_ref, ti_ref, fi_ref,
                embed_ref, wi_ref, wh_ref, b_ref, wd_ref, wout_ref, bout_ref,
                out_ref):
    pid = pl.program_id(0)
    si = si_ref[pid]
    ei = ei_ref[pid]

    iota_col = jax.lax.broadcasted_iota(jnp.int32, (_N, 1), 0)
    iota_row1 = jax.lax.broadcasted_iota(jnp.int32, (1, _N), 1)
    ip = (iota_col == si).astype(_F32)
    exit_row = iota_row1 == ei
    exit_col = (iota_col == ei).astype(_F32)

    iota_vcol = jax.lax.broadcasted_iota(jnp.int32, (_VOCAB, _N), 0)
    xs = []
    for tok in range(_L):
        drow = data_ref[0, tok:tok + 1, :]
        oh = (iota_vcol == drow).astype(_F32)
        xs.append(_mm_t(embed_ref[...], oh))

    ti_col = ti_ref[0]
    fi_col = fi_ref[0]
    iota_row = jax.lax.broadcasted_iota(jnp.int32, (_N, _N), 1)
    t_oh = (ti_col == iota_row).astype(_F32)
    f_oh = (fi_col == iota_row).astype(_F32)

    ones_row = jnp.ones((1, _N), _F32)

    c = [jnp.zeros((_H, _N), _F32) for _ in range(_LAYERS)]
    h = [jnp.zeros((_H, _N), _F32) for _ in range(_LAYERS)]

    for _step in range(_STEPS):
        cc = list(c)
        hh = list(h)
        for tok in range(_L):
            inp = xs[tok]
            for lay in range(_LAYERS):
                gates = (_mm_t(wi_ref[lay], inp) + _mm_t(wh_ref[lay], hh[lay])
                         + b_ref[lay])
                ig = _sigmoid(gates[0 * _H:1 * _H])
                fg = _sigmoid(gates[1 * _H:2 * _H])
                gg = jnp.tanh(gates[2 * _H:3 * _H])
                og = _sigmoid(gates[3 * _H:4 * _H])
                cc[lay] = fg * cc[lay] + ig * gg
                hh[lay] = og * jnp.tanh(cc[lay])
                inp = hh[lay]
        cm = [jnp.where(exit_row, c[lay], cc[lay]) for lay in range(_LAYERS)]
        hm = [jnp.where(exit_row, h[lay], hh[lay]) for lay in range(_LAYERS)]

        d = (_mm_t(cm[0], wd_ref[0 * _H:1 * _H]) + _mm_t(cm[1], wd_ref[1 * _H:2 * _H])
             + _mm_t(hm[0], wd_ref[2 * _H:3 * _H]) + _mm_t(hm[1], wd_ref[3 * _H:4 * _H])
             + bd_b_ref[0])
        pt = _sigmoid(d) * ip
        pf = ip - pt

        bm = t_oh * pt + f_oh * pf
        denom_row = _mm(ones_row, bm)
        inv = 1.0 / (denom_row + 1e-7)
        c = [_mm(cm[lay], bm) * inv for lay in range(_LAYERS)]
        h = [_mm(hm[lay], bm) * inv for lay in range(_LAYERS)]
        ip = _mm_t(bm, jnp.ones((_N, 1), _F32))

    ec = [_mm(c[lay], exit_col) for lay in range(_LAYERS)]
    eh = [_mm(h[lay], exit_col) for lay in range(_LAYERS)]
    out = (_mm_t(ec[0], wout_ref[0 * _H:1 * _H]) + _mm_t(ec[1], wout_ref[1 * _H:2 * _H])
           + _mm_t(eh[0], wout_ref[2 * _H:3 * _H]) + _mm_t(eh[1], wout_ref[3 * _H:4 * _H])
           + bout_ref[...])
    out_ref[0] = out


def kernel(data, true_branch_nodes, false_branch_nodes, start_index, exit_index,
           steps, embed, Wi, Wh, b, W_bd, b_bd, W_out, b_out):
    del steps
    data_t = jnp.transpose(data, (0, 2, 1)).astype(jnp.int32)
    ti3 = true_branch_nodes.astype(jnp.int32).reshape(_B, _N, 1)
    fi3 = false_branch_nodes.astype(jnp.int32).reshape(_B, _N, 1)
    wd = (W_bd[:, 0] - W_bd[:, 1]).reshape(4 * _H, 1)
    bd_b = (b_bd[0] - b_bd[1]).reshape(1)
    b2 = b.reshape(_LAYERS, 4 * _H, 1)
    bout2 = b_out.reshape(1, _OUT)
    si = start_index.astype(jnp.int32)
    ei = exit_index.astype(jnp.int32)

    out = pl.pallas_call(
        _fwd_kernel,
        grid=(_B,),
        in_specs=[
            pl.BlockSpec(memory_space=pltpu.SMEM),
            pl.BlockSpec(memory_space=pltpu.SMEM),
            pl.BlockSpec(memory_space=pltpu.SMEM),
            pl.BlockSpec((1, _L, _N), lambda bb: (bb, 0, 0)),
            pl.BlockSpec((1, _N, 1), lambda bb: (bb, 0, 0)),
            pl.BlockSpec((1, _N, 1), lambda bb: (bb, 0, 0)),
            pl.BlockSpec((_VOCAB, _H), lambda bb: (0, 0)),
            pl.BlockSpec((_LAYERS, _H, 4 * _H), lambda bb: (0, 0, 0)),
            pl.BlockSpec((_LAYERS, _H, 4 * _H), lambda bb: (0, 0, 0)),
            pl.BlockSpec((_LAYERS, 4 * _H, 1), lambda bb: (0, 0, 0)),
            pl.BlockSpec((4 * _H, 1), lambda bb: (0, 0)),
            pl.BlockSpec((4 * _H, _OUT), lambda bb: (0, 0)),
            pl.BlockSpec((1, _OUT), lambda bb: (0, 0)),
        ],
        out_specs=pl.BlockSpec((1, 1, _OUT), lambda bb: (bb, 0, 0)),
        out_shape=jax.ShapeDtypeStruct((_B, 1, _OUT), _F32),
        compiler_params=pltpu.CompilerParams(
            dimension_semantics=("arbitrary",)),
    )(si, ei, bd_b, data_t, ti3, fi3, embed, Wi, Wh, b2, wd, W_out, bout2)
    return out.reshape(_B, _OUT)


# speedup vs baseline: 58.4897x; 1.1296x over previous
import jax
import jax.numpy as jnp
from jax.experimental import pallas as pl
from jax.experimental.pallas import tpu as pltpu

_B = 8
_N = 512
_L = 4
_H = 64
_VOCAB = 1024
_OUT = 1000
_LAYERS = 2
_STEPS = 4

_F32 = jnp.float32


def _mm(a, b):
    return jax.lax.dot_general(a, b, (((1,), (0,)), ((), ())),
                               preferred_element_type=_F32)


def _mm_t(a, b):
    return jax.lax.dot_general(a, b, (((0,), (0,)), ((), ())),
                               preferred_element_type=_F32)


def _sigmoid_pre(y):
    return 0.5 * jnp.tanh(y) + 0.5


def _fwd_kernel(si_ref, ei_ref, bd_b_ref,
                data_ref, ti_ref, fi_ref,
                ew_ref, wh0_ref, w1_ref, b_ref, wd_ref, wout_ref, bout_ref,
                out_ref):
    pid = pl.program_id(0)
    si = si_ref[pid]
    ei = ei_ref[pid]

    iota_col = jax.lax.broadcasted_iota(jnp.int32, (_N, 1), 0)
    iota_row1 = jax.lax.broadcasted_iota(jnp.int32, (1, _N), 1)
    ip = (iota_col == si).astype(_F32)
    exit_row = iota_row1 == ei
    exit_col = (iota_col == ei).astype(_F32)

    iota_vcol = jax.lax.broadcasted_iota(jnp.int32, (_VOCAB, _N), 0)
    gx0 = []
    for tok in range(_L):
        drow = data_ref[0, tok:tok + 1, :]
        oh = (iota_vcol == drow).astype(_F32)
        gx0.append(_mm_t(ew_ref[...], oh))

    ti_col = ti_ref[0]
    fi_col = fi_ref[0]
    iota_row = jax.lax.broadcasted_iota(jnp.int32, (_N, _N), 1)
    t_oh = (ti_col == iota_row).astype(_F32)
    f_oh = (fi_col == iota_row).astype(_F32)

    ones_col = jnp.ones((_N, 1), _F32)
    ones_rows = jnp.ones((8, _N), _F32)

    s_prev = jnp.zeros((4 * _H, _N), _F32)

    for _step in range(_STEPS):
        c0 = s_prev[0 * _H:1 * _H]
        c1 = s_prev[1 * _H:2 * _H]
        h0 = s_prev[2 * _H:3 * _H]
        h1 = s_prev[3 * _H:4 * _H]
        for tok in range(_L):
            g0 = gx0[tok] + _mm_t(wh0_ref[...], h0) + b_ref[0]
            ig = _sigmoid_pre(g0[0 * _H:1 * _H])
            fg = _sigmoid_pre(g0[1 * _H:2 * _H])
            gg = jnp.tanh(g0[2 * _H:3 * _H])
            og = _sigmoid_pre(g0[3 * _H:4 * _H])
            c0 = fg * c0 + ig * gg
            h0 = og * jnp.tanh(c0)
            inp1 = jnp.concatenate([h0, h1], axis=0)
            g1 = _mm_t(w1_ref[...], inp1) + b_ref[1]
            ig = _sigmoid_pre(g1[0 * _H:1 * _H])
            fg = _sigmoid_pre(g1[1 * _H:2 * _H])
            gg = jnp.tanh(g1[2 * _H:3 * _H])
            og = _sigmoid_pre(g1[3 * _H:4 * _H])
            c1 = fg * c1 + ig * gg
            h1 = og * jnp.tanh(c1)
        s_new = jnp.concatenate([c0, c1, h0, h1], axis=0)
        sm = jnp.where(exit_row, s_prev, s_new)

        d = _mm_t(sm, wd_ref[...]) + bd_b_ref[0]
        pt = _sigmoid_pre(d) * ip
        pf = ip - pt

        bm = t_oh * pt + f_oh * pf
        sm_aug = jnp.concatenate([sm, ones_rows], axis=0)
        agg = _mm(sm_aug, bm)
        denom_row = agg[4 * _H:4 * _H + 1]
        inv = 1.0 / (denom_row + 1e-7)
        s_prev = agg[0:4 * _H] * inv
        ip = _mm_t(bm, ones_col)

    es = _mm(s_prev, exit_col)
    out = _mm_t(es, wout_ref[...]) + bout_ref[...]
    out_ref[0] = out


def kernel(data, true_branch_nodes, false_branch_nodes, start_index, exit_index,
           steps, embed, Wi, Wh, b, W_bd, b_bd, W_out, b_out):
    del steps
    data_t = jnp.transpose(data, (0, 2, 1)).astype(jnp.int32)
    ti3 = true_branch_nodes.astype(jnp.int32).reshape(_B, _N, 1)
    fi3 = false_branch_nodes.astype(jnp.int32).reshape(_B, _N, 1)
    scale = jnp.concatenate([jnp.full((2 * _H,), 0.5, _F32),
                             jnp.ones((_H,), _F32),
                             jnp.full((_H,), 0.5, _F32)])
    ew = embed @ (Wi[0] * scale)
    wh0 = Wh[0] * scale
    w1 = jnp.concatenate([Wi[1], Wh[1]], axis=0) * scale
    b2 = (b * scale).reshape(_LAYERS, 4 * _H, 1)
    wd = ((W_bd[:, 0] - W_bd[:, 1]) * 0.5).reshape(4 * _H, 1)
    bd_b = ((b_bd[0] - b_bd[1]) * 0.5).reshape(1)
    bout2 = b_out.reshape(1, _OUT)
    si = start_index.astype(jnp.int32)
    ei = exit_index.astype(jnp.int32)

    out = pl.pallas_call(
        _fwd_kernel,
        grid=(_B,),
        in_specs=[
            pl.BlockSpec(memory_space=pltpu.SMEM),
            pl.BlockSpec(memory_space=pltpu.SMEM),
            pl.BlockSpec(memory_space=pltpu.SMEM),
            pl.BlockSpec((1, _L, _N), lambda bb: (bb, 0, 0)),
            pl.BlockSpec((1, _N, 1), lambda bb: (bb, 0, 0)),
            pl.BlockSpec((1, _N, 1), lambda bb: (bb, 0, 0)),
            pl.BlockSpec((_VOCAB, 4 * _H), lambda bb: (0, 0)),
            pl.BlockSpec((_H, 4 * _H), lambda bb: (0, 0)),
            pl.BlockSpec((2 * _H, 4 * _H), lambda bb: (0, 0)),
            pl.BlockSpec((_LAYERS, 4 * _H, 1), lambda bb: (0, 0, 0)),
            pl.BlockSpec((4 * _H, 1), lambda bb: (0, 0)),
            pl.BlockSpec((4 * _H, _OUT), lambda bb: (0, 0)),
            pl.BlockSpec((1, _OUT), lambda bb: (0, 0)),
        ],
        out_specs=pl.BlockSpec((1, 1, _OUT), lambda bb: (bb, 0, 0)),
        out_shape=jax.ShapeDtypeStruct((_B, 1, _OUT), _F32),
        compiler_params=pltpu.CompilerParams(
            dimension_semantics=("arbitrary",)),
    )(si, ei, bd_b, data_t, ti3, fi3, ew, wh0, w1, b2, wd, W_out, bout2)
    return out.reshape(_B, _OUT)


# speedup vs baseline: 71.6331x; 1.2247x over previous
import jax
import jax.numpy as jnp
from jax.experimental import pallas as pl
from jax.experimental.pallas import tpu as pltpu

_B = 8
_N = 512
_L = 4
_H = 64
_VOCAB = 1024
_OUT = 1000
_LAYERS = 2
_STEPS = 4

_G = 4
_W = _G * _N

_F32 = jnp.float32


def _mm(a, b):
    return jax.lax.dot_general(a, b, (((1,), (0,)), ((), ())),
                               preferred_element_type=_F32)


def _mm_t(a, b):
    return jax.lax.dot_general(a, b, (((0,), (0,)), ((), ())),
                               preferred_element_type=_F32)


def _sigmoid_pre(y):
    return 0.5 * jnp.tanh(y) + 0.5


def _fwd_kernel(si_ref, ei_ref, bd_b_ref,
                data_ref, ti_ref, fi_ref,
                ew_ref, wh0_ref, w1_ref, b_ref, wd_ref, wout_ref, bout_ref,
                out_ref):
    pid = pl.program_id(0)

    iota_col = jax.lax.broadcasted_iota(jnp.int32, (_N, 1), 0)
    iota_row1 = jax.lax.broadcasted_iota(jnp.int32, (1, _N), 1)
    ips = [(iota_col == si_ref[pid * _G + g]).astype(_F32) for g in range(_G)]
    exit_row = jnp.concatenate(
        [iota_row1 == ei_ref[pid * _G + g] for g in range(_G)], axis=1)
    exit_cols = [(iota_col == ei_ref[pid * _G + g]).astype(_F32)
                 for g in range(_G)]

    iota_vcol = jax.lax.broadcasted_iota(jnp.int32, (_VOCAB, _W), 0)
    gx0 = []
    for tok in range(_L):
        drow = data_ref[tok:tok + 1, :]
        oh = (iota_vcol == drow).astype(_F32)
        gx0.append(_mm_t(ew_ref[...], oh))

    iota_row = jax.lax.broadcasted_iota(jnp.int32, (_N, _N), 1)
    t_oh = [(ti_ref[g] == iota_row).astype(_F32) for g in range(_G)]
    f_oh = [(fi_ref[g] == iota_row).astype(_F32) for g in range(_G)]

    ones_col = jnp.ones((_N, 1), _F32)
    ones_rows = jnp.ones((8, _N), _F32)

    s_prev = jnp.zeros((4 * _H, _W), _F32)

    for _step in range(_STEPS):
        c0 = s_prev[0 * _H:1 * _H]
        c1 = s_prev[1 * _H:2 * _H]
        h0 = s_prev[2 * _H:3 * _H]
        h1 = s_prev[3 * _H:4 * _H]
        for tok in range(_L):
            g0 = gx0[tok] + _mm_t(wh0_ref[...], h0) + b_ref[0]
            ig = _sigmoid_pre(g0[0 * _H:1 * _H])
            fg = _sigmoid_pre(g0[1 * _H:2 * _H])
            gg = jnp.tanh(g0[2 * _H:3 * _H])
            og = _sigmoid_pre(g0[3 * _H:4 * _H])
            c0 = fg * c0 + ig * gg
            h0 = og * jnp.tanh(c0)
            inp1 = jnp.concatenate([h0, h1], axis=0)
            g1 = _mm_t(w1_ref[...], inp1) + b_ref[1]
            ig = _sigmoid_pre(g1[0 * _H:1 * _H])
            fg = _sigmoid_pre(g1[1 * _H:2 * _H])
            gg = jnp.tanh(g1[2 * _H:3 * _H])
            og = _sigmoid_pre(g1[3 * _H:4 * _H])
            c1 = fg * c1 + ig * gg
            h1 = og * jnp.tanh(c1)
        s_new = jnp.concatenate([c0, c1, h0, h1], axis=0)
        sm = jnp.where(exit_row, s_prev, s_new)

        s_parts = []
        for g in range(_G):
            sm_g = sm[:, g * _N:(g + 1) * _N]
            d = _mm_t(sm_g, wd_ref[...]) + bd_b_ref[0]
            pt = _sigmoid_pre(d) * ips[g]
            pf = ips[g] - pt
            bm = t_oh[g] * pt + f_oh[g] * pf
            sm_aug = jnp.concatenate([sm_g, ones_rows], axis=0)
            agg = _mm(sm_aug, bm)
            inv = 1.0 / (agg[4 * _H:4 * _H + 1] + 1e-7)
            s_parts.append(agg[0:4 * _H] * inv)
            ips[g] = _mm_t(bm, ones_col)
        s_prev = jnp.concatenate(s_parts, axis=1)

    es = jnp.concatenate(
        [_mm(s_prev[:, g * _N:(g + 1) * _N], exit_cols[g]) for g in range(_G)],
        axis=1)
    out = _mm_t(es, wout_ref[...]) + bout_ref[...]
    out_ref[0] = out


def kernel(data, true_branch_nodes, false_branch_nodes, start_index, exit_index,
           steps, embed, Wi, Wh, b, W_bd, b_bd, W_out, b_out):
    del steps
    dm = jnp.transpose(data, (1, 0, 2)).reshape(_L, _B * _N)
    ti3 = true_branch_nodes.reshape(_B, _N, 1)
    fi3 = false_branch_nodes.reshape(_B, _N, 1)
    scale = jnp.concatenate([jnp.full((2 * _H,), 0.5, _F32),
                             jnp.ones((_H,), _F32),
                             jnp.full((_H,), 0.5, _F32)])
    ew = embed @ (Wi[0] * scale)
    wh0 = Wh[0] * scale
    w1 = jnp.concatenate([Wi[1], Wh[1]], axis=0) * scale
    b2 = (b * scale).reshape(_LAYERS, 4 * _H, 1)
    wd = ((W_bd[:, 0] - W_bd[:, 1]) * 0.5).reshape(4 * _H, 1)
    bd_b = ((b_bd[0] - b_bd[1]) * 0.5).reshape(1)
    bout2 = b_out.reshape(1, _OUT)
    si = start_index.astype(jnp.int32)
    ei = exit_index.astype(jnp.int32)

    out = pl.pallas_call(
        _fwd_kernel,
        grid=(_B // _G,),
        in_specs=[
            pl.BlockSpec(memory_space=pltpu.SMEM),
            pl.BlockSpec(memory_space=pltpu.SMEM),
            pl.BlockSpec(memory_space=pltpu.SMEM),
            pl.BlockSpec((_L, _W), lambda bb: (0, bb)),
            pl.BlockSpec((_G, _N, 1), lambda bb: (bb, 0, 0)),
            pl.BlockSpec((_G, _N, 1), lambda bb: (bb, 0, 0)),
            pl.BlockSpec((_VOCAB, 4 * _H), lambda bb: (0, 0)),
            pl.BlockSpec((_H, 4 * _H), lambda bb: (0, 0)),
            pl.BlockSpec((2 * _H, 4 * _H), lambda bb: (0, 0)),
            pl.BlockSpec((_LAYERS, 4 * _H, 1), lambda bb: (0, 0, 0)),
            pl.BlockSpec((4 * _H, 1), lambda bb: (0, 0)),
            pl.BlockSpec((4 * _H, _OUT), lambda bb: (0, 0)),
            pl.BlockSpec((1, _OUT), lambda bb: (0, 0)),
        ],
        out_specs=pl.BlockSpec((1, _G, _OUT), lambda bb: (bb, 0, 0)),
        out_shape=jax.ShapeDtypeStruct((_B // _G, _G, _OUT), _F32),
        compiler_params=pltpu.CompilerParams(
            dimension_semantics=("arbitrary",)),
    )(si, ei, bd_b, dm, ti3, fi3, ew, wh0, w1, b2, wd, W_out, bout2)
    return out.reshape(_B, _OUT)
